# Initial kernel scaffold; baseline (speedup 1.0000x reference)
import jax
import jax.numpy as jnp
from jax.experimental import pallas as pl


def kernel(feature_map, bboxes, sample_rois, gt_roi_loc, anchors, rpn_w, rpn_b, reg_w, reg_b, cls_w, cls_b, fc1_w, fc1_b, fc2_w, fc2_b, head_reg_w, head_reg_b, head_cls_w, head_cls_b, gt_roi_label):
    raise NotImplementedError("write your pallas kernel here")


# speedup vs baseline: 2.3403x; 2.3403x over previous
import numpy as np
import jax
import jax.numpy as jnp
from jax.experimental import pallas as pl
from jax.experimental.pallas import tpu as pltpu

IMG = 800.0
FS = 50
NC = 21
HW = FS * FS
WPAD = FS + 2
ROWS = FS * WPAD
XROWS = WPAD * WPAD + 8
A_PAD = 22528
R = 128
K1BLK = 1792
K2BLK = 1024


def _rpn_kernel(x_ref, w_ref, b_ref, hw_ref, hb_ref, y_ref, col_ref):
    p = pl.program_id(0)
    for t in range(9):
        dy, dx = t // 3, t % 3
        s = dy * WPAD + dx
        for half in range(2):
            @pl.when(p == half)
            def _(t=t, s=s, half=half):
                col_ref[:, t * 512:(t + 1) * 512] = (
                    x_ref[pl.ds(half * (ROWS // 2) + s, ROWS // 2), :])
    acc = jnp.dot(col_ref[...], w_ref[...], preferred_element_type=jnp.float32)
    h = jnp.maximum(acc + b_ref[...], 0.0)
    y_ref[0] = jnp.dot(h, hw_ref[...], preferred_element_type=jnp.float32) + hb_ref[...]


def _anchor_kernel(bb_ref, anc_ref, ploc_ref, pcls_ref, cls_o, loc_o, trpn_o):
    a0 = anc_ref[0]
    a1 = anc_ref[1]
    a2 = anc_ref[2]
    a3 = anc_ref[3]
    valid = (a0 >= 0) & (a1 >= 0) & (a2 <= IMG) & (a3 <= IMG)
    area_a = (a2 - a0) * (a3 - a1)
    ious = []
    for g in range(16):
        b0 = bb_ref[g, 0]
        b1 = bb_ref[g, 1]
        b2 = bb_ref[g, 2]
        b3 = bb_ref[g, 3]
        w = jnp.clip(jnp.minimum(a2, b2) - jnp.maximum(a0, b0), 0.0)
        h = jnp.clip(jnp.minimum(a3, b3) - jnp.maximum(a1, b1), 0.0)
        inter = w * h
        area_b = (b2 - b0) * (b3 - b1)
        iou = inter / (area_a + area_b - inter + 1e-9)
        ious.append(jnp.where(valid, iou, -1.0))
    iou_max = ious[0]
    for g in range(1, 16):
        iou_max = jnp.maximum(iou_max, ious[g])
    target = jnp.where(valid & (iou_max < 0.3), 0, -1)
    target = jnp.where(valid & (iou_max >= 0.7), 1, target)
    gt_match = jnp.zeros_like(valid)
    for g in range(16):
        gt_match = gt_match | (ious[g] == jnp.max(ious[g]))
    gt_match = gt_match & valid
    target = jnp.where(gt_match, 1, target)
    first = jnp.full_like(target, 16)
    for g in range(15, -1, -1):
        first = jnp.where(ious[g] == iou_max, g, first)
    m0 = jnp.zeros_like(area_a)
    mb = [m0, m0, m0, m0]
    for g in range(16):
        hit = first == g
        for c in range(4):
            mb[c] = jnp.where(hit, bb_ref[g, c], mb[c])
    aw = a2 - a0
    ah = a3 - a1
    ax = a0 + aw / 2
    ay = a1 + ah / 2
    gw = mb[2] - mb[0]
    gh = mb[3] - mb[1]
    gx = mb[0] + gw / 2
    gy = mb[1] + gh / 2
    floc = [(gx - ax) / aw, (gy - ay) / ah,
            jnp.log(gw / aw + 1e-9), jnp.log(gh / ah + 1e-9)]
    l0 = pcls_ref[0]
    l1 = pcls_ref[1]
    mx = jnp.maximum(l0, l1)
    s0 = l0 - mx
    s1 = l1 - mx
    lse = jnp.log(jnp.exp(s0) + jnp.exp(s1))
    mce = (target >= 0).astype(jnp.float32)
    lab1 = jnp.clip(target, 0) == 1
    nll = -(jnp.where(lab1, s1, s0) - lse)
    cls_loss = jnp.sum(nll * mce) / jnp.maximum(jnp.sum(mce), 1.0)
    m = (target > 0).astype(jnp.float32)
    loc_loss = 0.0
    for c in range(4):
        d = jnp.abs(jnp.where(valid, floc[c], 0.0) - ploc_ref[c])
        sl = (jnp.where(d < 0.5, d * d * 0.5, 0.0) +
              jnp.where(d > 0.5, d - 0.5, 0.0))
        loc_loss = loc_loss + jnp.sum(m * sl)
    n_reg = jnp.maximum(jnp.sum(m), 1.0)
    cls_o[...] = jnp.reshape(cls_loss, (1, 1))
    loc_o[...] = jnp.reshape(loc_loss, (1, 1))
    trpn_o[...] = jnp.reshape(cls_loss + 10.0 / n_reg * loc_loss, (1, 1))


def _idx_kernel(rois_ref, f_ref, iy_ref, ix_ref):
    r = rois_ref[...]
    f = f_ref[...]
    sc = 1.0 / 16
    x1 = r[:, 0:1] * sc
    y1 = r[:, 1:2] * sc
    x2 = r[:, 2:3] * sc
    y2 = r[:, 3:4] * sc
    xs = x1 + f * (x2 - x1)
    ys = y1 + f * (y2 - y1)
    ix_ref[...] = jnp.clip(jnp.floor(xs), 0, FS - 1).astype(jnp.int32)
    iy_ref[...] = jnp.clip(jnp.floor(ys), 0, FS - 1).astype(jnp.int32)


def _pool_kernel(iy_ref, ix_ref, feat_ref, out_ref):
    r = pl.program_id(0) * (R // 2) + pl.program_id(1)
    for by in range(7):
        iy0 = iy_ref[r, by * 2] * FS
        iy1 = iy_ref[r, by * 2 + 1] * FS
        for bx in range(7):
            ix0 = ix_ref[r, bx * 2]
            ix1 = ix_ref[r, bx * 2 + 1]
            v00 = feat_ref[pl.ds(iy0 + ix0, 1), 0, :]
            v01 = feat_ref[pl.ds(iy0 + ix1, 1), 0, :]
            v10 = feat_ref[pl.ds(iy1 + ix0, 1), 0, :]
            v11 = feat_ref[pl.ds(iy1 + ix1, 1), 0, :]
            m = jnp.maximum(jnp.maximum(v00, v01), jnp.maximum(v10, v11))
            out_ref[pl.ds(by * 7 + bx, 1), 0, :] = m


def _make_fc_kernel(nsteps):
    def _fc_kernel(x_ref, w_ref, b_ref, o_ref, acc_ref):
        k = pl.program_id(1)

        @pl.when(k == 0)
        def _():
            acc_ref[...] = jnp.zeros_like(acc_ref)

        acc_ref[...] += jnp.dot(x_ref[...], w_ref[...],
                                preferred_element_type=jnp.float32)

        @pl.when(k == nsteps - 1)
        def _():
            o_ref[...] = jnp.maximum(acc_ref[...] + b_ref[...], 0.0)
    return _fc_kernel


def _head_kernel(h_ref, rw_ref, rb_ref, cw_ref, cb_ref, gtl_ref, lab_ref,
                 cls_o, loc_o, troi_o):
    h = h_ref[...]
    loc = jnp.dot(h, rw_ref[...], preferred_element_type=jnp.float32) + rb_ref[...]
    cls = jnp.dot(h, cw_ref[...], preferred_element_type=jnp.float32) + cb_ref[...]
    lab = lab_ref[...]
    mx = jnp.max(cls, axis=1, keepdims=True)
    sh = cls - mx
    logp = sh - jnp.log(jnp.sum(jnp.exp(sh), axis=1, keepdims=True))
    i21 = jax.lax.broadcasted_iota(jnp.int32, (R, NC), 1)
    oh = (i21 == lab).astype(jnp.float32)
    cls_loss = -jnp.sum(logp * oh) / R
    i84 = jax.lax.broadcasted_iota(jnp.int32, (R, NC * 4), 1)
    spread = loc * (i84 // 4 == lab).astype(jnp.float32)
    smat = (jax.lax.broadcasted_iota(jnp.int32, (NC * 4, 4), 0) % 4 ==
            jax.lax.broadcasted_iota(jnp.int32, (NC * 4, 4), 1)).astype(jnp.float32)
    sel = jnp.dot(spread, smat, preferred_element_type=jnp.float32)
    x2 = jnp.abs(sel - gtl_ref[...])
    m2 = (lab > 0).astype(jnp.float32)
    sl = (jnp.where(x2 < 0.5, x2 * x2 * 0.5, 0.0) +
          jnp.where(x2 > 0.5, x2 - 0.5, 0.0))
    loc_loss = jnp.sum(m2 * sl)
    n2 = jnp.maximum(jnp.sum(m2), 1.0)
    cls_o[...] = jnp.reshape(cls_loss, (1, 1))
    loc_o[...] = jnp.reshape(loc_loss, (1, 1))
    troi_o[...] = jnp.reshape(cls_loss + 10.0 / n2 * loc_loss, (1, 1))


_SCALAR = jax.ShapeDtypeStruct((1, 1), jnp.float32)


def kernel(feature_map, bboxes, sample_rois, gt_roi_loc, anchors,
           rpn_w, rpn_b, reg_w, reg_b, cls_w, cls_b,
           fc1_w, fc1_b, fc2_w, fc2_b,
           head_reg_w, head_reg_b, head_cls_w, head_cls_b, gt_roi_label):
    f32 = jnp.float32
    xhwc = jnp.transpose(feature_map[0], (1, 2, 0))
    xpad = jnp.pad(xhwc, ((1, 1), (1, 1), (0, 0))).reshape(WPAD * WPAD, 512)
    xpad = jnp.pad(xpad, ((0, 8), (0, 0)))
    w9 = jnp.transpose(rpn_w, (2, 3, 1, 0)).reshape(9 * 512, 512)
    headw = jnp.concatenate([reg_w[:, :, 0, 0].T, cls_w[:, :, 0, 0].T], axis=1)
    headb = jnp.concatenate([reg_b, cls_b]).reshape(1, 54)

    y = pl.pallas_call(
        _rpn_kernel,
        grid=(2,),
        in_specs=[
            pl.BlockSpec((XROWS, 512), lambda p: (0, 0)),
            pl.BlockSpec((9 * 512, 512), lambda p: (0, 0)),
            pl.BlockSpec((1, 512), lambda p: (0, 0)),
            pl.BlockSpec((512, 54), lambda p: (0, 0)),
            pl.BlockSpec((1, 54), lambda p: (0, 0)),
        ],
        out_specs=pl.BlockSpec((1, ROWS // 2, 54), lambda p: (p, 0, 0)),
        out_shape=jax.ShapeDtypeStruct((2, ROWS // 2, 54), f32),
        scratch_shapes=[pltpu.VMEM((ROWS // 2, 9 * 512), f32)],
        compiler_params=pltpu.CompilerParams(
            dimension_semantics=(pltpu.PARALLEL,),
            vmem_limit_bytes=56 * 1024 * 1024),
    )(xpad, w9, rpn_b.reshape(1, 512), headw, headb)

    y2 = y.reshape(FS, WPAD, 54)[:, :FS, :]
    pred_loc = y2[..., :36].reshape(HW * 9, 4)
    pred_cls = y2[..., 36:54].reshape(HW * 9, 2)

    pad_anc = jnp.tile(jnp.array([[-8.0, -8.0, -4.0, -4.0]], f32),
                       (A_PAD - HW * 9, 1))
    anc_p = jnp.concatenate([anchors, pad_anc], axis=0)
    ploc_p = jnp.pad(pred_loc, ((0, A_PAD - HW * 9), (0, 0)))
    pcls_p = jnp.pad(pred_cls, ((0, A_PAD - HW * 9), (0, 0)))

    anct = anc_p.T.reshape(4, A_PAD // 128, 128)
    ploct = ploc_p.T.reshape(4, A_PAD // 128, 128)
    pclst = pcls_p.T.reshape(2, A_PAD // 128, 128)
    rpn_cls_loss, rpn_loc_loss, t_rpn = pl.pallas_call(
        _anchor_kernel,
        grid=(1,),
        in_specs=[
            pl.BlockSpec(memory_space=pltpu.SMEM),
            pl.BlockSpec((4, A_PAD // 128, 128), lambda p: (0, 0, 0)),
            pl.BlockSpec((4, A_PAD // 128, 128), lambda p: (0, 0, 0)),
            pl.BlockSpec((2, A_PAD // 128, 128), lambda p: (0, 0, 0)),
        ],
        out_specs=[pl.BlockSpec((1, 1), lambda p: (0, 0))] * 3,
        out_shape=[_SCALAR] * 3,
        compiler_params=pltpu.CompilerParams(
            dimension_semantics=(pltpu.ARBITRARY,)),
    )(bboxes, anct, ploct, pclst)

    fr = ((np.arange(7, dtype=np.float32)[:, None] +
           (np.arange(2, dtype=np.float32)[None, :] + np.float32(0.5)) / np.float32(2))
          / np.float32(7)).reshape(1, 14)
    iy, ix = pl.pallas_call(
        _idx_kernel,
        grid=(1,),
        in_specs=[pl.BlockSpec((R, 4), lambda p: (0, 0)),
                  pl.BlockSpec((1, 14), lambda p: (0, 0))],
        out_specs=[pl.BlockSpec((R, 14), lambda p: (0, 0))] * 2,
        out_shape=[jax.ShapeDtypeStruct((R, 14), jnp.int32)] * 2,
        compiler_params=pltpu.CompilerParams(
            dimension_semantics=(pltpu.ARBITRARY,)),
    )(sample_rois, jnp.asarray(fr))

    feat3 = xhwc.reshape(HW, 1, 512)
    p3 = pl.pallas_call(
        _pool_kernel,
        grid_spec=pltpu.PrefetchScalarGridSpec(
            num_scalar_prefetch=2,
            grid=(2, R // 2),
            in_specs=[pl.BlockSpec((HW, 1, 512), lambda p, j, *_: (0, 0, 0))],
            out_specs=pl.BlockSpec((49, 1, 512),
                                   lambda p, j, *_: (p * (R // 2) + j, 0, 0)),
        ),
        out_shape=jax.ShapeDtypeStruct((R * 49, 1, 512), f32),
        compiler_params=pltpu.CompilerParams(
            dimension_semantics=(pltpu.PARALLEL, pltpu.ARBITRARY)),
    )(iy, ix, feat3)

    pool = p3.reshape(R, 49, 512).transpose(0, 2, 1).reshape(R, 49 * 512)

    def fc(x, w, b, kblk):
        kk = w.shape[0] // kblk
        return pl.pallas_call(
            _make_fc_kernel(kk),
            grid=(2, kk),
            in_specs=[
                pl.BlockSpec((R, kblk), lambda n, k: (0, k)),
                pl.BlockSpec((kblk, 2048), lambda n, k: (k, n)),
                pl.BlockSpec((1, 2048), lambda n, k: (0, n)),
            ],
            out_specs=pl.BlockSpec((R, 2048), lambda n, k: (0, n)),
            out_shape=jax.ShapeDtypeStruct((R, 4096), f32),
            scratch_shapes=[pltpu.VMEM((R, 2048), f32)],
            compiler_params=pltpu.CompilerParams(
                dimension_semantics=(pltpu.PARALLEL, pltpu.ARBITRARY),
                vmem_limit_bytes=56 * 1024 * 1024),
        )(x, w, b.reshape(1, 4096))

    h1 = fc(pool, fc1_w, fc1_b, K1BLK)
    h2 = fc(h1, fc2_w, fc2_b, K2BLK)

    roi_cls_loss, roi_loc_loss, t_roi = pl.pallas_call(
        _head_kernel,
        grid=(1,),
        in_specs=[
            pl.BlockSpec((R, 4096), lambda p: (0, 0)),
            pl.BlockSpec((4096, 84), lambda p: (0, 0)),
            pl.BlockSpec((1, 84), lambda p: (0, 0)),
            pl.BlockSpec((4096, NC), lambda p: (0, 0)),
            pl.BlockSpec((1, NC), lambda p: (0, 0)),
            pl.BlockSpec((R, 4), lambda p: (0, 0)),
            pl.BlockSpec((R, 1), lambda p: (0, 0)),
        ],
        out_specs=[pl.BlockSpec((1, 1), lambda p: (0, 0))] * 3,
        out_shape=[_SCALAR] * 3,
        compiler_params=pltpu.CompilerParams(
            dimension_semantics=(pltpu.ARBITRARY,)),
    )(h2, head_reg_w, head_reg_b.reshape(1, 84),
      head_cls_w, head_cls_b.reshape(1, NC),
      gt_roi_loc, gt_roi_label.reshape(R, 1).astype(jnp.int32))

    t_loss = t_roi[0, 0] + t_rpn[0, 0]
    return (rpn_cls_loss[0, 0], rpn_loc_loss[0, 0],
            roi_cls_loss[0, 0], roi_loc_loss[0, 0], t_loss)


# speedup vs baseline: 2.4820x; 1.0605x over previous
import numpy as np
import jax
import jax.numpy as jnp
from jax.experimental import pallas as pl
from jax.experimental.pallas import tpu as pltpu

IMG = 800.0
FS = 50
NC = 21
HW = FS * FS
WPAD = FS + 2
ROWS = FS * WPAD
XROWS = WPAD * WPAD + 8
A_PAD = 22528
R = 128
K1BLK = 1792
K2BLK = 1024


def _rpn_kernel(x_ref, w_ref, b_ref, hw_ref, hb_ref, y_ref, col_ref):
    p = pl.program_id(0)
    for t in range(9):
        dy, dx = t // 3, t % 3
        s = dy * WPAD + dx
        for half in range(2):
            @pl.when(p == half)
            def _(t=t, s=s, half=half):
                col_ref[:, t * 512:(t + 1) * 512] = (
                    x_ref[pl.ds(half * (ROWS // 2) + s, ROWS // 2), :])
    acc = jnp.dot(col_ref[...], w_ref[...], preferred_element_type=jnp.float32)
    h = jnp.maximum(acc + b_ref[...], 0.0)
    y_ref[0] = jnp.dot(h, hw_ref[...], preferred_element_type=jnp.float32) + hb_ref[...]


def _anchor_kernel(bb_ref, anc_ref, ploc_ref, pcls_ref, cls_o, loc_o, trpn_o):
    a0 = anc_ref[0]
    a1 = anc_ref[1]
    a2 = anc_ref[2]
    a3 = anc_ref[3]
    valid = (a0 >= 0) & (a1 >= 0) & (a2 <= IMG) & (a3 <= IMG)
    area_a = (a2 - a0) * (a3 - a1)
    ious = []
    for g in range(16):
        b0 = bb_ref[g, 0]
        b1 = bb_ref[g, 1]
        b2 = bb_ref[g, 2]
        b3 = bb_ref[g, 3]
        w = jnp.clip(jnp.minimum(a2, b2) - jnp.maximum(a0, b0), 0.0)
        h = jnp.clip(jnp.minimum(a3, b3) - jnp.maximum(a1, b1), 0.0)
        inter = w * h
        area_b = (b2 - b0) * (b3 - b1)
        iou = inter / (area_a + area_b - inter + 1e-9)
        ious.append(jnp.where(valid, iou, -1.0))
    iou_max = ious[0]
    for g in range(1, 16):
        iou_max = jnp.maximum(iou_max, ious[g])
    target = jnp.where(valid & (iou_max < 0.3), 0, -1)
    target = jnp.where(valid & (iou_max >= 0.7), 1, target)
    gt_match = jnp.zeros_like(valid)
    for g in range(16):
        gt_match = gt_match | (ious[g] == jnp.max(ious[g]))
    gt_match = gt_match & valid
    target = jnp.where(gt_match, 1, target)
    first = jnp.full_like(target, 16)
    for g in range(15, -1, -1):
        first = jnp.where(ious[g] == iou_max, g, first)
    m0 = jnp.zeros_like(area_a)
    mb = [m0, m0, m0, m0]
    for g in range(16):
        hit = first == g
        for c in range(4):
            mb[c] = jnp.where(hit, bb_ref[g, c], mb[c])
    aw = a2 - a0
    ah = a3 - a1
    ax = a0 + aw / 2
    ay = a1 + ah / 2
    gw = mb[2] - mb[0]
    gh = mb[3] - mb[1]
    gx = mb[0] + gw / 2
    gy = mb[1] + gh / 2
    floc = [(gx - ax) / aw, (gy - ay) / ah,
            jnp.log(gw / aw + 1e-9), jnp.log(gh / ah + 1e-9)]
    l0 = pcls_ref[0]
    l1 = pcls_ref[1]
    mx = jnp.maximum(l0, l1)
    s0 = l0 - mx
    s1 = l1 - mx
    lse = jnp.log(jnp.exp(s0) + jnp.exp(s1))
    mce = (target >= 0).astype(jnp.float32)
    lab1 = jnp.clip(target, 0) == 1
    nll = -(jnp.where(lab1, s1, s0) - lse)
    cls_loss = jnp.sum(nll * mce) / jnp.maximum(jnp.sum(mce), 1.0)
    m = (target > 0).astype(jnp.float32)
    loc_loss = 0.0
    for c in range(4):
        d = jnp.abs(jnp.where(valid, floc[c], 0.0) - ploc_ref[c])
        sl = (jnp.where(d < 0.5, d * d * 0.5, 0.0) +
              jnp.where(d > 0.5, d - 0.5, 0.0))
        loc_loss = loc_loss + jnp.sum(m * sl)
    n_reg = jnp.maximum(jnp.sum(m), 1.0)
    cls_o[...] = jnp.reshape(cls_loss, (1, 1))
    loc_o[...] = jnp.reshape(loc_loss, (1, 1))
    trpn_o[...] = jnp.reshape(cls_loss + 10.0 / n_reg * loc_loss, (1, 1))


def _idx_kernel(rois_ref, f_ref, iy_ref, ix_ref):
    r = rois_ref[...]
    f = f_ref[...]
    sc = 1.0 / 16
    x1 = r[:, 0:1] * sc
    y1 = r[:, 1:2] * sc
    x2 = r[:, 2:3] * sc
    y2 = r[:, 3:4] * sc
    xs = x1 + f * (x2 - x1)
    ys = y1 + f * (y2 - y1)
    ix_ref[...] = jnp.clip(jnp.floor(xs), 0, FS - 1).astype(jnp.int32)
    iy_ref[...] = jnp.clip(jnp.floor(ys), 0, FS - 1).astype(jnp.int32)


def _pool_kernel(iy_ref, ix_ref, feat_ref, out_ref):
    r0 = (pl.program_id(0) * (R // 16) + pl.program_id(1)) * 8
    for q in range(8):
        for by in range(7):
            iy0 = iy_ref[r0 + q, by * 2] * FS
            iy1 = iy_ref[r0 + q, by * 2 + 1] * FS
            for bx in range(7):
                ix0 = ix_ref[r0 + q, bx * 2]
                ix1 = ix_ref[r0 + q, bx * 2 + 1]
                v00 = feat_ref[pl.ds(iy0 + ix0, 1), 0, :]
                v01 = feat_ref[pl.ds(iy0 + ix1, 1), 0, :]
                v10 = feat_ref[pl.ds(iy1 + ix0, 1), 0, :]
                v11 = feat_ref[pl.ds(iy1 + ix1, 1), 0, :]
                m = jnp.maximum(jnp.maximum(v00, v01), jnp.maximum(v10, v11))
                out_ref[pl.ds(q * 49 + by * 7 + bx, 1), 0, :] = m


def _make_fc_kernel(nsteps):
    def _fc_kernel(x_ref, w_ref, b_ref, o_ref, acc_ref):
        k = pl.program_id(1)

        @pl.when(k == 0)
        def _():
            acc_ref[...] = jnp.zeros_like(acc_ref)

        acc_ref[...] += jnp.dot(x_ref[...], w_ref[...],
                                preferred_element_type=jnp.float32)

        @pl.when(k == nsteps - 1)
        def _():
            o_ref[...] = jnp.maximum(acc_ref[...] + b_ref[...], 0.0)
    return _fc_kernel


def _head_kernel(h_ref, rw_ref, rb_ref, cw_ref, cb_ref, gtl_ref, lab_ref,
                 cls_o, loc_o, troi_o):
    h = h_ref[...]
    loc = jnp.dot(h, rw_ref[...], preferred_element_type=jnp.float32) + rb_ref[...]
    cls = jnp.dot(h, cw_ref[...], preferred_element_type=jnp.float32) + cb_ref[...]
    lab = lab_ref[...]
    mx = jnp.max(cls, axis=1, keepdims=True)
    sh = cls - mx
    logp = sh - jnp.log(jnp.sum(jnp.exp(sh), axis=1, keepdims=True))
    i21 = jax.lax.broadcasted_iota(jnp.int32, (R, NC), 1)
    oh = (i21 == lab).astype(jnp.float32)
    cls_loss = -jnp.sum(logp * oh) / R
    i84 = jax.lax.broadcasted_iota(jnp.int32, (R, NC * 4), 1)
    spread = loc * (i84 // 4 == lab).astype(jnp.float32)
    smat = (jax.lax.broadcasted_iota(jnp.int32, (NC * 4, 4), 0) % 4 ==
            jax.lax.broadcasted_iota(jnp.int32, (NC * 4, 4), 1)).astype(jnp.float32)
    sel = jnp.dot(spread, smat, preferred_element_type=jnp.float32)
    x2 = jnp.abs(sel - gtl_ref[...])
    m2 = (lab > 0).astype(jnp.float32)
    sl = (jnp.where(x2 < 0.5, x2 * x2 * 0.5, 0.0) +
          jnp.where(x2 > 0.5, x2 - 0.5, 0.0))
    loc_loss = jnp.sum(m2 * sl)
    n2 = jnp.maximum(jnp.sum(m2), 1.0)
    cls_o[...] = jnp.reshape(cls_loss, (1, 1))
    loc_o[...] = jnp.reshape(loc_loss, (1, 1))
    troi_o[...] = jnp.reshape(cls_loss + 10.0 / n2 * loc_loss, (1, 1))


_SCALAR = jax.ShapeDtypeStruct((1, 1), jnp.float32)


def kernel(feature_map, bboxes, sample_rois, gt_roi_loc, anchors,
           rpn_w, rpn_b, reg_w, reg_b, cls_w, cls_b,
           fc1_w, fc1_b, fc2_w, fc2_b,
           head_reg_w, head_reg_b, head_cls_w, head_cls_b, gt_roi_label):
    f32 = jnp.float32
    xhwc = jnp.transpose(feature_map[0], (1, 2, 0))
    xpad = jnp.pad(xhwc, ((1, 1), (1, 1), (0, 0))).reshape(WPAD * WPAD, 512)
    xpad = jnp.pad(xpad, ((0, 8), (0, 0)))
    w9 = jnp.transpose(rpn_w, (2, 3, 1, 0)).reshape(9 * 512, 512)
    headw = jnp.concatenate([reg_w[:, :, 0, 0].T, cls_w[:, :, 0, 0].T], axis=1)
    headb = jnp.concatenate([reg_b, cls_b]).reshape(1, 54)

    y = pl.pallas_call(
        _rpn_kernel,
        grid=(2,),
        in_specs=[
            pl.BlockSpec((XROWS, 512), lambda p: (0, 0)),
            pl.BlockSpec((9 * 512, 512), lambda p: (0, 0)),
            pl.BlockSpec((1, 512), lambda p: (0, 0)),
            pl.BlockSpec((512, 54), lambda p: (0, 0)),
            pl.BlockSpec((1, 54), lambda p: (0, 0)),
        ],
        out_specs=pl.BlockSpec((1, ROWS // 2, 54), lambda p: (p, 0, 0)),
        out_shape=jax.ShapeDtypeStruct((2, ROWS // 2, 54), f32),
        scratch_shapes=[pltpu.VMEM((ROWS // 2, 9 * 512), f32)],
        compiler_params=pltpu.CompilerParams(
            dimension_semantics=(pltpu.PARALLEL,),
            vmem_limit_bytes=56 * 1024 * 1024),
    )(xpad, w9, rpn_b.reshape(1, 512), headw, headb)

    y2 = y.reshape(FS, WPAD, 54)[:, :FS, :]
    pred_loc = y2[..., :36].reshape(HW * 9, 4)
    pred_cls = y2[..., 36:54].reshape(HW * 9, 2)

    pad_anc = jnp.tile(jnp.array([[-8.0, -8.0, -4.0, -4.0]], f32),
                       (A_PAD - HW * 9, 1))
    anc_p = jnp.concatenate([anchors, pad_anc], axis=0)
    ploc_p = jnp.pad(pred_loc, ((0, A_PAD - HW * 9), (0, 0)))
    pcls_p = jnp.pad(pred_cls, ((0, A_PAD - HW * 9), (0, 0)))

    anct = anc_p.T.reshape(4, A_PAD // 128, 128)
    ploct = ploc_p.T.reshape(4, A_PAD // 128, 128)
    pclst = pcls_p.T.reshape(2, A_PAD // 128, 128)
    rpn_cls_loss, rpn_loc_loss, t_rpn = pl.pallas_call(
        _anchor_kernel,
        grid=(1,),
        in_specs=[
            pl.BlockSpec(memory_space=pltpu.SMEM),
            pl.BlockSpec((4, A_PAD // 128, 128), lambda p: (0, 0, 0)),
            pl.BlockSpec((4, A_PAD // 128, 128), lambda p: (0, 0, 0)),
            pl.BlockSpec((2, A_PAD // 128, 128), lambda p: (0, 0, 0)),
        ],
        out_specs=[pl.BlockSpec((1, 1), lambda p: (0, 0))] * 3,
        out_shape=[_SCALAR] * 3,
        compiler_params=pltpu.CompilerParams(
            dimension_semantics=(pltpu.ARBITRARY,)),
    )(bboxes, anct, ploct, pclst)

    fr = ((np.arange(7, dtype=np.float32)[:, None] +
           (np.arange(2, dtype=np.float32)[None, :] + np.float32(0.5)) / np.float32(2))
          / np.float32(7)).reshape(1, 14)
    iy, ix = pl.pallas_call(
        _idx_kernel,
        grid=(1,),
        in_specs=[pl.BlockSpec((R, 4), lambda p: (0, 0)),
                  pl.BlockSpec((1, 14), lambda p: (0, 0))],
        out_specs=[pl.BlockSpec((R, 14), lambda p: (0, 0))] * 2,
        out_shape=[jax.ShapeDtypeStruct((R, 14), jnp.int32)] * 2,
        compiler_params=pltpu.CompilerParams(
            dimension_semantics=(pltpu.ARBITRARY,)),
    )(sample_rois, jnp.asarray(fr))

    feat3 = xhwc.reshape(HW, 1, 512)
    p3 = pl.pallas_call(
        _pool_kernel,
        grid_spec=pltpu.PrefetchScalarGridSpec(
            num_scalar_prefetch=2,
            grid=(2, R // 16),
            in_specs=[pl.BlockSpec((HW, 1, 512), lambda p, j, *_: (0, 0, 0))],
            out_specs=pl.BlockSpec((8 * 49, 1, 512),
                                   lambda p, j, *_: (p * (R // 16) + j, 0, 0)),
        ),
        out_shape=jax.ShapeDtypeStruct((R * 49, 1, 512), f32),
        compiler_params=pltpu.CompilerParams(
            dimension_semantics=(pltpu.PARALLEL, pltpu.ARBITRARY)),
    )(iy, ix, feat3)

    pool = p3.reshape(R, 49, 512).transpose(0, 2, 1).reshape(R, 49 * 512)

    def fc(x, w, b, kblk):
        kk = w.shape[0] // kblk
        return pl.pallas_call(
            _make_fc_kernel(kk),
            grid=(2, kk),
            in_specs=[
                pl.BlockSpec((R, kblk), lambda n, k: (0, k)),
                pl.BlockSpec((kblk, 2048), lambda n, k: (k, n)),
                pl.BlockSpec((1, 2048), lambda n, k: (0, n)),
            ],
            out_specs=pl.BlockSpec((R, 2048), lambda n, k: (0, n)),
            out_shape=jax.ShapeDtypeStruct((R, 4096), f32),
            scratch_shapes=[pltpu.VMEM((R, 2048), f32)],
            compiler_params=pltpu.CompilerParams(
                dimension_semantics=(pltpu.PARALLEL, pltpu.ARBITRARY),
                vmem_limit_bytes=56 * 1024 * 1024),
        )(x, w, b.reshape(1, 4096))

    h1 = fc(pool, fc1_w, fc1_b, K1BLK)
    h2 = fc(h1, fc2_w, fc2_b, K2BLK)

    roi_cls_loss, roi_loc_loss, t_roi = pl.pallas_call(
        _head_kernel,
        grid=(1,),
        in_specs=[
            pl.BlockSpec((R, 4096), lambda p: (0, 0)),
            pl.BlockSpec((4096, 84), lambda p: (0, 0)),
            pl.BlockSpec((1, 84), lambda p: (0, 0)),
            pl.BlockSpec((4096, NC), lambda p: (0, 0)),
            pl.BlockSpec((1, NC), lambda p: (0, 0)),
            pl.BlockSpec((R, 4), lambda p: (0, 0)),
            pl.BlockSpec((R, 1), lambda p: (0, 0)),
        ],
        out_specs=[pl.BlockSpec((1, 1), lambda p: (0, 0))] * 3,
        out_shape=[_SCALAR] * 3,
        compiler_params=pltpu.CompilerParams(
            dimension_semantics=(pltpu.ARBITRARY,)),
    )(h2, head_reg_w, head_reg_b.reshape(1, 84),
      head_cls_w, head_cls_b.reshape(1, NC),
      gt_roi_loc, gt_roi_label.reshape(R, 1).astype(jnp.int32))

    t_loss = t_roi[0, 0] + t_rpn[0, 0]
    return (rpn_cls_loss[0, 0], rpn_loc_loss[0, 0],
            roi_cls_loss[0, 0], roi_loc_loss[0, 0], t_loss)
